# baseline (speedup 1.0000x reference)
import jax
import jax.numpy as jnp
from jax.experimental import pallas as pl
from jax.experimental.pallas import tpu as pltpu

_F32 = jnp.float32
_BF16 = jnp.bfloat16


def _rowshift(a, d):
    n = a.shape[0]
    z = jnp.zeros((abs(d), a.shape[1]), a.dtype)
    if d > 0:
        return jnp.concatenate([a[d:], z], axis=0)
    return jnp.concatenate([z, a[:n + d]], axis=0)


def _laneshift(a, d):
    z = jnp.zeros((a.shape[0], d), a.dtype)
    return jnp.concatenate([a[:, d:], z], axis=1)


def _backbone_kernel(x_ref, g1_ref, g2_ref, g3_ref, b1_ref, b2_ref, b3_ref,
                     rsel_ref, mfc_ref, fcb_ref, out_ref):
  for i in range(2):
    x = x_ref[0, i]
    x3 = jnp.concatenate([_rowshift(x, -1), x, _rowshift(x, 1)],
                         axis=1).astype(_BF16)
    y1 = jnp.dot(x3, g1_ref[...], preferred_element_type=_F32) + b1_ref[...]
    y1 = jnp.maximum(y1, 0.0)
    m = jnp.maximum(y1, _rowshift(y1, 1))
    p1 = jnp.maximum(m, _laneshift(m, 16))

    def conv_block(p, hstep, g_ref, b_ref):
        zl = jnp.zeros((128, 128), _F32)
        zr = jnp.zeros((128, 384), _F32)
        ppad = jnp.concatenate([zl, p, zr], axis=1).astype(_BF16)
        parts = []
        for j in range(8):
            t = jnp.dot(ppad[:, 256 * j:256 * j + 512], g_ref[j],
                        preferred_element_type=_F32)
            parts.append(_rowshift(t[:, :256], -hstep) + t[:, 256:512]
                         + _rowshift(t[:, 512:], hstep))
        y = jnp.concatenate(parts, axis=1) + b_ref[...]
        return jnp.maximum(y, 0.0)

    y2 = conv_block(p1, 2, g2_ref, b2_ref)
    m2 = jnp.maximum(y2, _rowshift(y2, 2))
    p2 = jnp.maximum(m2, _laneshift(m2, 32))
    y3 = conv_block(p2, 4, g3_ref, b3_ref)
    rs = jnp.dot(rsel_ref[...], y3, preferred_element_type=_F32)
    feat = jnp.dot(rs, mfc_ref[...], preferred_element_type=_F32) + fcb_ref[...]
    out_ref[0, 0, i:i + 1, :] = feat


def _head_kernel(fq_ref, fkv_ref, qw_ref, kw_ref, vw_ref, out_ref):
    cdims = (((1,), (1,)), ((), ()))
    q = jax.lax.dot_general(fq_ref[...], qw_ref[...], cdims,
                            preferred_element_type=_F32) * 0.125
    k = jax.lax.dot_general(fkv_ref[...], kw_ref[...], cdims,
                            preferred_element_type=_F32)
    v = jax.lax.dot_general(fkv_ref[...], vw_ref[...], cdims,
                            preferred_element_type=_F32)
    scores = jnp.concatenate(
        [jnp.sum(q * k[32 * s:32 * s + 32], axis=1, keepdims=True)
         for s in range(6)], axis=1)
    mx = jnp.max(scores, axis=1, keepdims=True)
    e = jnp.exp(scores - mx)
    a = e / jnp.sum(e, axis=1, keepdims=True)
    ctx = sum(a[:, s:s + 1] * v[32 * s:32 * s + 32] for s in range(6))
    out_ref[...] = ctx


def _band_matrix(w, win, cin, cout):
    eyes = jnp.stack([jnp.eye(win, k=1 - d, dtype=_F32) for d in range(3)])
    g5 = jnp.einsum('ocbd,duv->bucvo', w, eyes)
    full = jnp.zeros((3, 2 * win, cin, win, cout), _F32)
    full = full.at[:, 0::2].set(g5)
    return full.reshape(3, 2048, 2048)


def _band_slices(full):
    pad = jnp.pad(full, ((0, 0), (128, 384), (0, 0)))
    slices = []
    for j in range(8):
        sl = pad[:, 256 * j:256 * j + 512, 256 * j:256 * j + 256]
        slices.append(sl.transpose(1, 0, 2).reshape(512, 768))
    return jnp.stack(slices).astype(_BF16)


def kernel(bev_seq, conv1_w, conv1_b, conv2_w, conv2_b, conv3_w, conv3_b,
           fc_w, fc_b, q_w, k_w, v_w):
    B, T, C, H, W = bev_seq.shape
    xs = bev_seq.reshape(B, T, H, W)

    eyes = jnp.stack([jnp.eye(128, k=1 - d, dtype=_F32) for d in range(3)])
    g1 = jnp.einsum('obd,duv->buvo', conv1_w[:, 0],
                    eyes).reshape(384, 2048).astype(_BF16)
    g2 = _band_slices(_band_matrix(conv2_w, 64, 16, 32))
    g3 = _band_slices(_band_matrix(conv3_w, 32, 32, 64))
    b1 = jnp.tile(conv1_b, 128)[None, :]
    b2 = jnp.tile(conv2_b, 64)[None, :]
    b3 = jnp.tile(conv3_b, 32)[None, :]
    rsel = (jnp.arange(128) % 4 == 0).astype(_F32)[None, :]
    mfc = (jnp.tile(fc_w.T, (32, 1)) / 1024.0)
    fcb = fc_b[None, :]

    const = lambda *shape: pl.BlockSpec(
        shape, lambda b, r: tuple(0 for _ in shape))
    feats = pl.pallas_call(
        _backbone_kernel,
        grid=(B, 4),
        in_specs=[
            pl.BlockSpec((1, 2, H, W), lambda b, r: (b, 4 + r, 0, 0)),
            const(384, 2048),
            const(8, 512, 768),
            const(8, 512, 768),
            const(1, 2048),
            const(1, 2048),
            const(1, 2048),
            const(1, 128),
            const(2048, 128),
            const(1, 128),
        ],
        out_specs=pl.BlockSpec((1, 1, 2, 128), lambda b, r: (b, r, 0, 0)),
        out_shape=jax.ShapeDtypeStruct((B, 4, 2, 128), _F32),
        compiler_params=pltpu.CompilerParams(
            dimension_semantics=("parallel", "arbitrary"),
            vmem_limit_bytes=64 * 1024 * 1024,
        ),
    )(xs, g1, g2, g3, b1, b2, b3, rsel, mfc, fcb)

    f_all = feats.reshape(B, 8, 128)[:, 1:8, :]
    fq = f_all[:, 6]
    fkv = f_all[:, :6].transpose(1, 0, 2).reshape(6 * B, 128)

    out = pl.pallas_call(
        _head_kernel,
        out_shape=jax.ShapeDtypeStruct((B, 128), _F32),
    )(fq, fkv, q_w, k_w, v_w)
    return out


# speedup vs baseline: 1.0401x; 1.0401x over previous
import jax
import jax.numpy as jnp
from jax.experimental import pallas as pl
from jax.experimental.pallas import tpu as pltpu

_F32 = jnp.float32
_BF16 = jnp.bfloat16


def _rowshift(a, d):
    n = a.shape[0]
    z = jnp.zeros((abs(d), a.shape[1]), a.dtype)
    if d > 0:
        return jnp.concatenate([a[d:], z], axis=0)
    return jnp.concatenate([z, a[:n + d]], axis=0)


def _laneshift(a, d):
    z = jnp.zeros((a.shape[0], d), a.dtype)
    return jnp.concatenate([a[:, d:], z], axis=1)


def _backbone_kernel(x_ref, g1_ref, g2_ref, g3_ref, b1_ref, b2_ref, b3_ref,
                     rsel_ref, mfc_ref, fcb_ref, out_ref):
  if True:
    x = x_ref[0]
    x3 = jnp.concatenate([_rowshift(x, -1), x, _rowshift(x, 1)],
                         axis=1).astype(_BF16)
    y1 = jnp.dot(x3, g1_ref[...], preferred_element_type=_F32) + b1_ref[...]
    y1 = jnp.maximum(y1, 0.0)
    m = jnp.maximum(y1, _rowshift(y1, 1))
    p1 = jnp.maximum(m, _laneshift(m, 16))

    def conv_block(p, hstep, g_ref, b_ref):
        zl = jnp.zeros((128, 128), _F32)
        zr = jnp.zeros((128, 384), _F32)
        ppad = jnp.concatenate([zl, p, zr], axis=1).astype(_BF16)
        parts = []
        for j in range(8):
            t = jnp.dot(ppad[:, 256 * j:256 * j + 512], g_ref[j],
                        preferred_element_type=_F32)
            parts.append(_rowshift(t[:, :256], -hstep) + t[:, 256:512]
                         + _rowshift(t[:, 512:], hstep))
        y = jnp.concatenate(parts, axis=1) + b_ref[...]
        return jnp.maximum(y, 0.0)

    y2 = conv_block(p1, 2, g2_ref, b2_ref)
    m2 = jnp.maximum(y2, _rowshift(y2, 2))
    p2 = jnp.maximum(m2, _laneshift(m2, 32))
    y3 = conv_block(p2, 4, g3_ref, b3_ref)
    rs = jnp.dot(rsel_ref[...], y3, preferred_element_type=_F32)
    feat = jnp.dot(rs, mfc_ref[...], preferred_element_type=_F32) + fcb_ref[...]
    out_ref[0] = feat


def _head_kernel(fq_ref, fkv_ref, qw_ref, kw_ref, vw_ref, out_ref):
    cdims = (((1,), (1,)), ((), ()))
    q = jax.lax.dot_general(fq_ref[...], qw_ref[...], cdims,
                            preferred_element_type=_F32) * 0.125
    k = jax.lax.dot_general(fkv_ref[...], kw_ref[...], cdims,
                            preferred_element_type=_F32)
    v = jax.lax.dot_general(fkv_ref[...], vw_ref[...], cdims,
                            preferred_element_type=_F32)
    scores = jnp.concatenate(
        [jnp.sum(q * k[32 * s:32 * s + 32], axis=1, keepdims=True)
         for s in range(6)], axis=1)
    mx = jnp.max(scores, axis=1, keepdims=True)
    e = jnp.exp(scores - mx)
    a = e / jnp.sum(e, axis=1, keepdims=True)
    ctx = sum(a[:, s:s + 1] * v[32 * s:32 * s + 32] for s in range(6))
    out_ref[...] = ctx


def _band_matrix(w, win, cin, cout):
    eyes = jnp.stack([jnp.eye(win, k=1 - d, dtype=_F32) for d in range(3)])
    g5 = jnp.einsum('ocbd,duv->bucvo', w, eyes)
    full = jnp.zeros((3, 2 * win, cin, win, cout), _F32)
    full = full.at[:, 0::2].set(g5)
    return full.reshape(3, 2048, 2048)


def _band_slices(full):
    pad = jnp.pad(full, ((0, 0), (128, 384), (0, 0)))
    slices = []
    for j in range(8):
        sl = pad[:, 256 * j:256 * j + 512, 256 * j:256 * j + 256]
        slices.append(sl.transpose(1, 0, 2).reshape(512, 768))
    return jnp.stack(slices).astype(_BF16)


def kernel(bev_seq, conv1_w, conv1_b, conv2_w, conv2_b, conv3_w, conv3_b,
           fc_w, fc_b, q_w, k_w, v_w):
    B, T, C, H, W = bev_seq.shape
    xs = bev_seq.reshape(B * T, H, W)

    eyes = jnp.stack([jnp.eye(128, k=1 - d, dtype=_F32) for d in range(3)])
    g1 = jnp.einsum('obd,duv->buvo', conv1_w[:, 0],
                    eyes).reshape(384, 2048).astype(_BF16)
    g2 = _band_slices(_band_matrix(conv2_w, 64, 16, 32))
    g3 = _band_slices(_band_matrix(conv3_w, 32, 32, 64))
    b1 = jnp.tile(conv1_b, 128)[None, :]
    b2 = jnp.tile(conv2_b, 64)[None, :]
    b3 = jnp.tile(conv3_b, 32)[None, :]
    rsel = (jnp.arange(128) % 4 == 0).astype(_F32)[None, :]
    mfc = (jnp.tile(fc_w.T, (32, 1)) / 1024.0)
    fcb = fc_b[None, :]

    const = lambda *shape: pl.BlockSpec(
        shape, lambda b, r: tuple(0 for _ in shape))
    feats = pl.pallas_call(
        _backbone_kernel,
        grid=(B, 7),
        in_specs=[
            pl.BlockSpec((1, H, W), lambda b, r: (T * b + 9 + r, 0, 0)),
            const(384, 2048),
            const(8, 512, 768),
            const(8, 512, 768),
            const(1, 2048),
            const(1, 2048),
            const(1, 2048),
            const(1, 128),
            const(2048, 128),
            const(1, 128),
        ],
        out_specs=pl.BlockSpec((1, 1, 128), lambda b, r: (7 * b + r, 0, 0)),
        out_shape=jax.ShapeDtypeStruct((B * 7, 1, 128), _F32),
        compiler_params=pltpu.CompilerParams(
            dimension_semantics=("parallel", "arbitrary"),
            vmem_limit_bytes=64 * 1024 * 1024,
        ),
    )(xs, g1, g2, g3, b1, b2, b3, rsel, mfc, fcb)

    f_all = feats[:, 0, :].reshape(B, 7, 128)
    fq = f_all[:, 6]
    fkv = f_all[:, :6].transpose(1, 0, 2).reshape(6 * B, 128)

    out = pl.pallas_call(
        _head_kernel,
        out_shape=jax.ShapeDtypeStruct((B, 128), _F32),
    )(fq, fkv, q_w, k_w, v_w)
    return out
